# baseline (speedup 1.0000x reference)
import functools
import jax
import jax.numpy as jnp
from jax import lax
from jax.experimental import pallas as pl
from jax.experimental.pallas import tpu as pltpu
from jax.experimental.pallas import tpu_sc as plsc

NC = 2
NS = 16
L = 16
NW = NC * NS
KSEG = 1024
SR = 16


def _pool(B, C, H, W):
  P = B * C
  PPW = P // NW
  NPASS = PPW // 2
  NSTRIP = H // SR
  GPR = W // L
  mesh = plsc.VectorSubcoreMesh(core_axis_name="c", subcore_axis_name="s")

  @functools.partial(
      pl.kernel,
      mesh=mesh,
      out_type=jax.ShapeDtypeStruct((P * KSEG,), jnp.float32),
      compiler_params=pltpu.CompilerParams(
          needs_layout_passes=False, use_tc_tiling_on_sc=False
      ),
      scratch_types=[
          pltpu.VMEM((2, SR, W), jnp.int32),
          pltpu.VMEM((2, 2, SR, W), jnp.float32),
          pltpu.VMEM((L * KSEG,), jnp.float32),
          pltpu.VMEM((L * KSEG,), jnp.float32),
          pltpu.VMEM((L * KSEG,), jnp.float32),
          pltpu.VMEM((L * KSEG,), jnp.float32),
          pltpu.VMEM((KSEG,), jnp.float32),
          pltpu.SemaphoreType.DMA,
          pltpu.SemaphoreType.DMA,
      ],
  )
  def k(img_hbm, spx_hbm, out_hbm, lbl_v, d_v,
        a00_v, a01_v, a10_v, a11_v, row_v, sem0, sem1):
    wid = lax.axis_index("s") * NC + lax.axis_index("c")
    lane = lax.iota(jnp.int32, L)
    lane_k = lane * KSEG
    neg_inf = jnp.full((L,), -jnp.inf, jnp.float32)
    sems = (sem0, sem1)

    def issue(s, slot, b, c0):
      r0 = s * SR
      pltpu.async_copy(
          spx_hbm.at[b, pl.ds(r0, SR), :], lbl_v.at[slot], sems[slot])
      pltpu.async_copy(
          img_hbm.at[b, pl.ds(c0, 2), pl.ds(r0, SR), :], d_v.at[slot],
          sems[slot])

    def wait(slot):
      pltpu.make_async_copy(
          spx_hbm.at[0, pl.ds(0, SR), :], lbl_v.at[slot], sems[slot]).wait()
      pltpu.make_async_copy(
          img_hbm.at[0, pl.ds(0, 2), pl.ds(0, SR), :], d_v.at[slot],
          sems[slot]).wait()

    for ps in range(NPASS):
      p0 = wid * PPW + 2 * ps
      b = p0 // C
      c0 = p0 - b * C

      def init_body(j, _):
        o = j * (4 * L)
        for u in range(4):
          a00_v[pl.ds(o + u * L, L)] = neg_inf
          a01_v[pl.ds(o + u * L, L)] = neg_inf
          a10_v[pl.ds(o + u * L, L)] = neg_inf
          a11_v[pl.ds(o + u * L, L)] = neg_inf
        return 0

      lax.fori_loop(0, KSEG // 4, init_body, 0)

      issue(0, 0, b, c0)

      def process(slot):
        def row_body(r, _):
          for g in range(GPR):
            o = g * L
            acc_a = a00_v if g % 2 == 0 else a01_v
            acc_b = a10_v if g % 2 == 0 else a11_v
            lbl = lbl_v[slot, r, pl.ds(o, L)]
            idx = lane_k + lbl
            v0 = d_v[slot, 0, r, pl.ds(o, L)]
            v1 = d_v[slot, 1, r, pl.ds(o, L)]
            ca = plsc.load_gather(acc_a, [idx])
            cb = plsc.load_gather(acc_b, [idx])
            plsc.store_scatter(acc_a, [idx], jnp.maximum(ca, v0))
            plsc.store_scatter(acc_b, [idx], jnp.maximum(cb, v1))
          return 0

        lax.fori_loop(0, SR, row_body, 0)

      def strip_body(s2, _):
        s = s2 * 2
        issue(s + 1, 1, b, c0)
        wait(0)
        process(0)

        @pl.when(s2 + 1 < NSTRIP // 2)
        def _():
          issue(s + 2, 0, b, c0)

        wait(1)
        process(1)
        return 0

      lax.fori_loop(0, NSTRIP // 2, strip_body, 0)

      def fin0_body(j, _):
        m = jnp.maximum(a00_v[pl.ds(j * L, L)], a01_v[pl.ds(j * L, L)])
        for l in range(1, L):
          m = jnp.maximum(m, a00_v[pl.ds(l * KSEG + j * L, L)])
          m = jnp.maximum(m, a01_v[pl.ds(l * KSEG + j * L, L)])
        row_v[pl.ds(j * L, L)] = m
        return 0

      lax.fori_loop(0, KSEG // L, fin0_body, 0)
      pltpu.sync_copy(row_v, out_hbm.at[pl.ds(p0 * KSEG, KSEG)])

      def fin1_body(j, _):
        m = jnp.maximum(a10_v[pl.ds(j * L, L)], a11_v[pl.ds(j * L, L)])
        for l in range(1, L):
          m = jnp.maximum(m, a10_v[pl.ds(l * KSEG + j * L, L)])
          m = jnp.maximum(m, a11_v[pl.ds(l * KSEG + j * L, L)])
        row_v[pl.ds(j * L, L)] = m
        return 0

      lax.fori_loop(0, KSEG // L, fin1_body, 0)
      pltpu.sync_copy(row_v, out_hbm.at[pl.ds((p0 + 1) * KSEG, KSEG)])

  return k


@jax.jit
def kernel(img, spx):
  B, C, H, W = img.shape
  out = _pool(B, C, H, W)(img, spx.astype(jnp.int32))
  return out.reshape(B, C, KSEG)
